# baseline (speedup 1.0000x reference)
import functools

import jax
import jax.numpy as jnp
from jax import lax
from jax.experimental import pallas as pl
from jax.experimental.pallas import tpu as pltpu
from jax.experimental.pallas import tpu_sc as plsc

_N = 50000
_D = 256
_B = 256
_NW = 32
_C = 1568
_NP = _NW * _C
_CH = 224
_NCH = _C // _CH
_L = 16

_F32 = jnp.float32
_NEG = -1e30


def _col(v):
    ib = (lax.broadcasted_iota(jnp.int32, (_B, _B), 0) ==
          lax.broadcasted_iota(jnp.int32, (_B, _B), 1))
    return jnp.sum(jnp.where(ib, v, 0.0), axis=1, keepdims=True)


def _tc_body(m_ref, s_ref, r_ref, h_ref, c_ref, wih_ref, whh_ref, bias_ref,
             hn_ref, cn_ref, q_ref):
    mp = m_ref[...]
    sp = s_ref[...]
    mstar = jnp.max(mp, axis=0, keepdims=True)
    valid = mp > -1e29
    w = jnp.where(valid, jnp.exp(mp - mstar), 0.0)
    sstar = jnp.sum(w * sp, axis=0, keepdims=True)
    rstar = jnp.zeros((_B, _D), _F32)
    for i in range(_NW):
        wcol = _col(w[i:i + 1])
        ri = r_ref[pl.ds(i * _B, _B), :]
        ri = jnp.where(wcol > 0.0, ri, 0.0)
        rstar = rstar + wcol * ri
    r_fin = rstar / (_col(sstar) + 1e-16)
    h = h_ref[...]
    q_star = jnp.concatenate([h, r_fin], axis=1)
    g = (lax.dot_general(q_star, wih_ref[...], (((1,), (1,)), ((), ())),
                         preferred_element_type=_F32)
         + lax.dot_general(h, whh_ref[...], (((1,), (1,)), ((), ())),
                           preferred_element_type=_F32)
         + bias_ref[...])
    gi = jax.nn.sigmoid(g[:, :_D])
    gf = jax.nn.sigmoid(g[:, _D:2 * _D])
    gg = jnp.tanh(g[:, 2 * _D:3 * _D])
    go = jax.nn.sigmoid(g[:, 3 * _D:])
    c_new = gf * c_ref[...] + gi * gg
    h_new = go * jnp.tanh(c_new)
    hn_ref[...] = h_new
    cn_ref[...] = c_new
    q_ref[...] = q_star


def _tc_step(mp, sp, rp, h, c, W_ih, W_hh, bias):
    return pl.pallas_call(
        _tc_body,
        out_shape=[
            jax.ShapeDtypeStruct((_B, _D), _F32),
            jax.ShapeDtypeStruct((_B, _D), _F32),
            jax.ShapeDtypeStruct((_B, 2 * _D), _F32),
        ],
    )(mp, sp, rp, h, c, W_ih, W_hh, bias)


def _lanesum(v):
    iota = lax.broadcasted_iota(jnp.int32, (_L,), 0)
    for k in (8, 4, 2, 1):
        idx = jnp.bitwise_xor(iota, k)
        v = v + v.at[idx].get(mode="promise_in_bounds")
    return v


def _sc_attn(x, bat, h):
    mesh = plsc.VectorSubcoreMesh(core_axis_name="c", subcore_axis_name="s")

    @functools.partial(
        pl.kernel,
        mesh=mesh,
        out_type=[
            jax.ShapeDtypeStruct((_NW, _B), _F32),
            jax.ShapeDtypeStruct((_NW, _B), _F32),
            jax.ShapeDtypeStruct((_NW * _B, _D), _F32),
        ],
        scratch_types=[
            pltpu.VMEM((_CH, _D), _F32),
            pltpu.VMEM((_CH + _L,), jnp.int32),
            pltpu.VMEM((_L, _D), _F32),
            pltpu.VMEM((1, _D), _F32),
            pltpu.VMEM((_B,), _F32),
            pltpu.VMEM((_B,), _F32),
            pltpu.VMEM((_L,), jnp.int32),
            pltpu.SemaphoreType.DMA,
        ],
    )
    def k(x_hbm, bat_hbm, h_hbm, m_out, s_out, r_out,
          xbuf, bbuf, hcur, rcur, mloc, sloc, idxb, sem):
        cid = lax.axis_index("c")
        sid = lax.axis_index("s")
        wid = sid * 2 + cid
        start = wid * _C
        iota = lax.broadcasted_iota(jnp.int32, (_L,), 0)

        for j in range(_B // _L):
            mloc[pl.ds(j * _L, _L)] = jnp.full((_L,), _NEG, _F32)
            sloc[pl.ds(j * _L, _L)] = jnp.zeros((_L,), _F32)
        for j in range(_D // _L):
            rcur[0, pl.ds(j * _L, _L)] = jnp.zeros((_L,), _F32)

        def bat_at(i):
            return bbuf[pl.ds(i, _L)][0]

        def load_h(b):
            pltpu.async_copy(h_hbm.at[jnp.full((_L,), b, jnp.int32)],
                             hcur, sem).wait()

        def flush(b_old, m_old, s_old):
            for j in range(_B // _L):
                sl = pl.ds(j * _L, _L)
                sel = (iota + j * _L) == b_old
                mloc[sl] = jnp.where(sel, m_old, mloc[sl])
                sloc[sl] = jnp.where(sel, s_old, sloc[sl])
            idxb[pl.ds(0, _L)] = jnp.where(iota == 0, wid * _B + b_old,
                                           idxb[pl.ds(0, _L)])
            pltpu.async_copy(rcur, r_out.at[idxb.at[pl.ds(0, 1)]],
                             sem).wait()

        def row_body(i, carry):
            b_cur, m, s_v = carry
            bi = bat_at(i)

            def on_new(c):
                b_old, m_old = c
                flush(b_old, m_old, s_v)
                load_h(bi)
                return bi, jnp.asarray(_NEG, _F32)

            b_cur, m = lax.cond(bi != b_cur, on_new, lambda c: c,
                                (b_cur, m))

            acc = jnp.zeros((_L,), _F32)
            for j in range(_D // _L):
                sl = pl.ds(j * _L, _L)
                acc = acc + xbuf[i, sl] * hcur[0, sl]
            e = _lanesum(acc)[0]

            m_new = lax.max(m, e)
            alpha = jnp.exp(jnp.full((_L,), m - m_new, _F32))
            p = jnp.exp(jnp.full((_L,), e - m_new, _F32))
            s_v = s_v * alpha + p
            for j in range(_D // _L):
                sl = pl.ds(j * _L, _L)
                rcur[0, sl] = rcur[0, sl] * alpha + p * xbuf[i, sl]
            return (b_cur, m_new, s_v)

        def chunk_body(kk, carry):
            base = pl.multiple_of(start + kk * _CH, _CH)
            pltpu.sync_copy(bat_hbm.at[pl.ds(base, _CH)],
                            bbuf.at[pl.ds(0, _CH)])
            pltpu.sync_copy(x_hbm.at[pl.ds(base, _CH)], xbuf)
            rows = jnp.clip(_N - base, 0, _CH)
            return lax.fori_loop(0, rows, row_body, carry)

        pltpu.sync_copy(bat_hbm.at[pl.ds(start, _CH)], bbuf.at[pl.ds(0, _CH)])
        b0 = bat_at(0)
        load_h(b0)
        carry0 = (b0, jnp.asarray(_NEG, _F32), jnp.zeros((_L,), _F32))
        b_cur, m, s_v = lax.fori_loop(0, _NCH, chunk_body, carry0)

        flush(b_cur, m, s_v)
        pltpu.sync_copy(mloc, m_out.at[wid])
        pltpu.sync_copy(sloc, s_out.at[wid])

    return k(x, bat, h)


def kernel(x, batch, W_ih, W_hh, b_ih, b_hh):
    xp = jnp.pad(x, ((0, _NP - _N), (0, 0)))
    batp = jnp.pad(batch, (0, _NP - _N))
    bias = (b_ih + b_hh).reshape(1, 4 * _D)
    h = jnp.zeros((_B, _D), _F32)
    c = jnp.zeros((_B, _D), _F32)
    mp = jnp.full((_NW, _B), _NEG, _F32)
    sp = jnp.zeros((_NW, _B), _F32)
    rp = jnp.zeros((_NW * _B, _D), _F32)
    q_star = None
    for _ in range(3):
        h, c, q_star = _tc_step(mp, sp, rp, h, c, W_ih, W_hh, bias)
        mp, sp, rp = _sc_attn(xp, batp, h)
    _, _, q_star = _tc_step(mp, sp, rp, h, c, W_ih, W_hh, bias)
    return q_star


# speedup vs baseline: 1.5604x; 1.5604x over previous
import functools

import jax
import jax.numpy as jnp
from jax import lax
from jax.experimental import pallas as pl
from jax.experimental.pallas import tpu as pltpu
from jax.experimental.pallas import tpu_sc as plsc

_N = 50000
_D = 256
_B = 256
_NW = 32
_C = 1568
_NP = _NW * _C
_CH = 224
_NCH = _C // _CH
_L = 16
_NG = _CH // _L

_F32 = jnp.float32
_NEG = -1e30


def _col(v):
    ib = (lax.broadcasted_iota(jnp.int32, (_B, _B), 0) ==
          lax.broadcasted_iota(jnp.int32, (_B, _B), 1))
    return jnp.sum(jnp.where(ib, v, 0.0), axis=1, keepdims=True)


def _tc_body(m_ref, s_ref, r_ref, h_ref, c_ref, wih_ref, whh_ref, bias_ref,
             hn_ref, cn_ref, q_ref):
    mp = m_ref[...]
    sp = s_ref[...]
    mstar = jnp.max(mp, axis=0, keepdims=True)
    valid = mp > -1e29
    w = jnp.where(valid, jnp.exp(mp - mstar), 0.0)
    sstar = jnp.sum(w * sp, axis=0, keepdims=True)
    rstar = jnp.zeros((_B, _D), _F32)
    for i in range(_NW):
        wcol = _col(w[i:i + 1])
        ri = r_ref[pl.ds(i * _B, _B), :]
        ri = jnp.where(wcol > 0.0, ri, 0.0)
        rstar = rstar + wcol * ri
    r_fin = rstar / (_col(sstar) + 1e-16)
    h = h_ref[...]
    q_star = jnp.concatenate([h, r_fin], axis=1)
    g = (lax.dot_general(q_star, wih_ref[...], (((1,), (1,)), ((), ())),
                         preferred_element_type=_F32)
         + lax.dot_general(h, whh_ref[...], (((1,), (1,)), ((), ())),
                           preferred_element_type=_F32)
         + bias_ref[...])
    gi = jax.nn.sigmoid(g[:, :_D])
    gf = jax.nn.sigmoid(g[:, _D:2 * _D])
    gg = jnp.tanh(g[:, 2 * _D:3 * _D])
    go = jax.nn.sigmoid(g[:, 3 * _D:])
    c_new = gf * c_ref[...] + gi * gg
    h_new = go * jnp.tanh(c_new)
    hn_ref[...] = h_new
    cn_ref[...] = c_new
    q_ref[...] = q_star


def _tc_step(mp, sp, rp, h, c, W_ih, W_hh, bias):
    return pl.pallas_call(
        _tc_body,
        out_shape=[
            jax.ShapeDtypeStruct((_B, _D), _F32),
            jax.ShapeDtypeStruct((_B, _D), _F32),
            jax.ShapeDtypeStruct((_B, 2 * _D), _F32),
        ],
    )(mp, sp, rp, h, c, W_ih, W_hh, bias)


_IOTA = None


def _swap(v, k):
    iota = lax.broadcasted_iota(jnp.int32, (_L,), 0)
    return v.at[jnp.bitwise_xor(iota, k)].get(mode="promise_in_bounds")


def _lanesum(v):
    for k in (8, 4, 2, 1):
        v = v + _swap(v, k)
    return v


def _lanemax(v):
    for k in (8, 4, 2, 1):
        v = jnp.maximum(v, _swap(v, k))
    return v


def _tree16(vs):
    cur = list(vs)
    for k in (8, 4, 2, 1):
        n = len(cur) // 2
        iota = lax.broadcasted_iota(jnp.int32, (_L,), 0)
        sel = (iota & k) == 0
        cur = [jnp.where(sel, cur[i] + _swap(cur[i], k),
                         cur[i + n] + _swap(cur[i + n], k))
               for i in range(n)]
    return cur[0]


def _splat(v, lane):
    return v.at[jnp.full((_L,), lane, jnp.int32)].get(
        mode="promise_in_bounds")


def _sc_attn(x, bat, h):
    mesh = plsc.VectorSubcoreMesh(core_axis_name="c", subcore_axis_name="s")

    @functools.partial(
        pl.kernel,
        mesh=mesh,
        out_type=[
            jax.ShapeDtypeStruct((_NW, _B), _F32),
            jax.ShapeDtypeStruct((_NW, _B), _F32),
            jax.ShapeDtypeStruct((_NW * _B + 8, _D), _F32),
        ],
        scratch_types=[
            pltpu.VMEM((_CH, _D), _F32),
            pltpu.VMEM((_CH + _L,), jnp.int32),
            pltpu.VMEM((_L, _D), _F32),
            pltpu.VMEM((1, _D), _F32),
            pltpu.VMEM((_L,), _F32),
            pltpu.VMEM((_L,), _F32),
            pltpu.VMEM((_B,), _F32),
            pltpu.VMEM((_B,), _F32),
            pltpu.VMEM((_L,), jnp.int32),
            pltpu.SemaphoreType.DMA,
        ],
    )
    def k(x_hbm, bat_hbm, h_hbm, m_out, s_out, r_out,
          xbuf, bbuf, hcur, rcur, mvec, svec, mloc, sloc, idxb, sem):
        cid = lax.axis_index("c")
        sid = lax.axis_index("s")
        wid = sid * 2 + cid
        start = wid * _C
        iota = lax.broadcasted_iota(jnp.int32, (_L,), 0)

        for j in range(_B // _L):
            mloc[pl.ds(j * _L, _L)] = jnp.full((_L,), _NEG, _F32)
            sloc[pl.ds(j * _L, _L)] = jnp.zeros((_L,), _F32)
        for j in range(_D // _L):
            rcur[0, pl.ds(j * _L, _L)] = jnp.zeros((_L,), _F32)
        mvec[...] = jnp.full((_L,), _NEG, _F32)
        svec[...] = jnp.zeros((_L,), _F32)

        def load_h(b):
            bsafe = jnp.minimum(b, _B - 1)
            pltpu.async_copy(h_hbm.at[jnp.full((_L,), bsafe, jnp.int32)],
                             hcur, sem).wait()

        def flush(b_old):
            m_old = mvec[...]
            s_old = svec[...]
            for j in range(_B // _L):
                sl = pl.ds(j * _L, _L)
                sel = (iota + j * _L) == b_old
                mloc[sl] = jnp.where(sel, m_old, mloc[sl])
                sloc[sl] = jnp.where(sel, s_old, sloc[sl])
            ridx = jnp.where(b_old < _B, wid * _B + b_old, _NW * _B)
            idxb[pl.ds(0, _L)] = jnp.where(iota == 0, ridx,
                                           idxb[pl.ds(0, _L)])
            pltpu.async_copy(rcur, r_out.at[idxb.at[pl.ds(0, 1)]],
                             sem).wait()

        def fast_group(gbase):
            accs = []
            for r in range(_L):
                acc = jnp.zeros((_L,), _F32)
                for j in range(_D // _L):
                    sl = pl.ds(j * _L, _L)
                    acc = acc + xbuf[gbase + r, sl] * hcur[0, sl]
                accs.append(acc)
            e_vec = _tree16(accs)
            m_grp = _lanemax(e_vec)
            p_vec = jnp.exp(e_vec - m_grp)
            s_grp = _lanesum(p_vec)
            m_v = mvec[...]
            m_new = jnp.maximum(m_v, m_grp)
            alpha = jnp.exp(m_v - m_new)
            beta = jnp.exp(m_grp - m_new)
            mvec[...] = m_new
            svec[...] = svec[...] * alpha + s_grp * beta
            ps = [_splat(p_vec, r) for r in range(_L)]
            for j in range(_D // _L):
                sl = pl.ds(j * _L, _L)
                t = jnp.zeros((_L,), _F32)
                for r in range(_L):
                    t = t + ps[r] * xbuf[gbase + r, sl]
                rcur[0, sl] = rcur[0, sl] * alpha + beta * t

        def slow_group(gbase, bv, b_cur):
            for r in range(_L):
                bi = bv[r]
                ch = bi != b_cur

                def on_new(b):
                    flush(b)
                    load_h(bi)
                    return bi

                b_cur = lax.cond(ch, on_new, lambda b: b, b_cur)
                chf = jnp.where(ch, 1.0, 0.0)
                m_old = mvec[...]
                m_v = m_old + chf * (_NEG - m_old)
                acc = jnp.zeros((_L,), _F32)
                for j in range(_D // _L):
                    sl = pl.ds(j * _L, _L)
                    acc = acc + xbuf[gbase + r, sl] * hcur[0, sl]
                e_v = _lanesum(acc)
                m_new = jnp.maximum(m_v, e_v)
                alpha = jnp.exp(m_v - m_new)
                p = jnp.exp(e_v - m_new)
                mvec[...] = m_new
                svec[...] = svec[...] * alpha + p
                for j in range(_D // _L):
                    sl = pl.ds(j * _L, _L)
                    rcur[0, sl] = rcur[0, sl] * alpha + p * xbuf[gbase + r, sl]
            return b_cur

        def group_body(g, b_cur):
            gbase = g * _L
            bv = bbuf[pl.ds(gbase, _L)]
            b_first = bv[0]
            b_last = bv[_L - 1]
            fast = jnp.logical_and(b_first == b_cur, b_last == b_cur)

            def do_fast(b):
                fast_group(gbase)
                return b

            def do_slow(b):
                return slow_group(gbase, bv, b)

            return lax.cond(fast, do_fast, do_slow, b_cur)

        def chunk_body(kk, b_cur):
            base = pl.multiple_of(start + kk * _CH, _CH)
            pltpu.sync_copy(bat_hbm.at[pl.ds(base, _CH)],
                            bbuf.at[pl.ds(0, _CH)])
            pltpu.sync_copy(x_hbm.at[pl.ds(base, _CH)], xbuf)
            return lax.fori_loop(0, _NG, group_body, b_cur)

        pltpu.sync_copy(bat_hbm.at[pl.ds(start, _CH)], bbuf.at[pl.ds(0, _CH)])
        b0 = bbuf[pl.ds(0, _L)][0]
        load_h(b0)
        b_cur = lax.fori_loop(0, _NCH, chunk_body, b0)

        flush(b_cur)
        pltpu.sync_copy(mloc, m_out.at[wid])
        pltpu.sync_copy(sloc, s_out.at[wid])

    return k(x, bat, h)


def kernel(x, batch, W_ih, W_hh, b_ih, b_hh):
    xp = jnp.pad(x, ((0, _NP - _N), (0, 0)))
    batp = jnp.pad(batch, (0, _NP - _N), constant_values=_B)
    bias = (b_ih + b_hh).reshape(1, 4 * _D)
    h = jnp.zeros((_B, _D), _F32)
    c = jnp.zeros((_B, _D), _F32)
    mp = jnp.full((_NW, _B), _NEG, _F32)
    sp = jnp.zeros((_NW, _B), _F32)
    rp = jnp.zeros((_NW * _B + 8, _D), _F32)
    q_star = None
    for _ in range(3):
        h, c, q_star = _tc_step(mp, sp, rp[:_NW * _B], h, c,
                                W_ih, W_hh, bias)
        mp, sp, rp = _sc_attn(xp, batp, h)
    _, _, q_star = _tc_step(mp, sp, rp[:_NW * _B], h, c, W_ih, W_hh, bias)
    return q_star


# speedup vs baseline: 3.1421x; 2.0136x over previous
import functools

import jax
import jax.numpy as jnp
from jax import lax
from jax.experimental import pallas as pl
from jax.experimental.pallas import tpu as pltpu
from jax.experimental.pallas import tpu_sc as plsc

_N = 50000
_D = 256
_B = 256

_NW = 32
_C = 448
_CH = 224
_NCH = _C // _CH
_L = 16
_NG = _CH // _L
_NSC = _NW * _C
_NTC = _N - _NSC
_R = 2048
_NBLK = (_NTC + _R - 1) // _R
_NPTC = _NBLK * _R

_F32 = jnp.float32
_NEG = -1e30


def _col(v):
    ib = (lax.broadcasted_iota(jnp.int32, (_B, _B), 0) ==
          lax.broadcasted_iota(jnp.int32, (_B, _B), 1))
    return jnp.sum(jnp.where(ib, v, 0.0), axis=1, keepdims=True)


def _tca_body(bat_ref, x_ref, h_ref, m_out, s_out, r_out, m_s, s_s, r_s):
    blk = pl.program_id(0)

    @pl.when(blk == 0)
    def _init():
        m_s[...] = jnp.full_like(m_s, _NEG)
        s_s[...] = jnp.zeros_like(s_s)
        r_s[...] = jnp.zeros_like(r_s)

    bat = bat_ref[0]
    iota_b = lax.broadcasted_iota(jnp.int32, (_R, _B), 1)
    pmask = bat == iota_b
    pf = pmask.astype(_F32)
    h = h_ref[...]
    h_hi = h.astype(jnp.bfloat16).astype(_F32)
    h_lo = h - h_hi
    qg = (lax.dot_general(pf, h_hi, (((1,), (0,)), ((), ())),
                          preferred_element_type=_F32)
          + lax.dot_general(pf, h_lo, (((1,), (0,)), ((), ())),
                            preferred_element_type=_F32))
    xb = x_ref[...]
    e = jnp.sum(xb * qg, axis=1, keepdims=True)
    em = jnp.where(pmask, e, _NEG)
    mblk = jnp.max(em, axis=0, keepdims=True)
    m_old = m_s[...]
    m_new = jnp.maximum(m_old, mblk)
    scale = jnp.exp(m_old - m_new)
    gm = jnp.sum(jnp.where(pmask, m_new, 0.0), axis=1, keepdims=True)
    ex = jnp.exp(e - gm)
    pw = pf * ex
    sblk = jnp.sum(pw, axis=0, keepdims=True)
    m_s[...] = m_new
    s_s[...] = s_s[...] * scale + sblk
    scale_col = _col(scale)
    racc = lax.dot_general(pw, xb, (((0,), (0,)), ((), ())),
                           preferred_element_type=_F32)
    r_s[...] = r_s[...] * scale_col + racc

    @pl.when(blk == _NBLK - 1)
    def _fin():
        m_out[...] = m_s[...]
        s_out[...] = s_s[...]
        r_out[...] = r_s[...]


def _tc_attn(bat3, xtc, h):
    return pl.pallas_call(
        _tca_body,
        grid=(_NBLK,),
        in_specs=[
            pl.BlockSpec((1, _R, 1), lambda b: (b, 0, 0)),
            pl.BlockSpec((_R, _D), lambda b: (b, 0)),
            pl.BlockSpec((_B, _D), lambda b: (0, 0)),
        ],
        out_specs=[
            pl.BlockSpec((1, _B), lambda b: (0, 0)),
            pl.BlockSpec((1, _B), lambda b: (0, 0)),
            pl.BlockSpec((_B, _D), lambda b: (0, 0)),
        ],
        out_shape=[
            jax.ShapeDtypeStruct((1, _B), _F32),
            jax.ShapeDtypeStruct((1, _B), _F32),
            jax.ShapeDtypeStruct((_B, _D), _F32),
        ],
        scratch_shapes=[
            pltpu.VMEM((1, _B), _F32),
            pltpu.VMEM((1, _B), _F32),
            pltpu.VMEM((_B, _D), _F32),
        ],
    )(bat3, xtc, h)


def _ml_body(m_ref, s_ref, r_ref, mt_ref, st_ref, rt_ref, h_ref, c_ref,
             wih_ref, whh_ref, bias_ref, hn_ref, cn_ref, q_ref):
    mp = m_ref[...]
    sp = s_ref[...]
    mt = mt_ref[...]
    mstar = jnp.maximum(jnp.max(mp, axis=0, keepdims=True), mt)
    valid = mp > -1e29
    w = jnp.where(valid, jnp.exp(mp - mstar), 0.0)
    wt = jnp.where(mt > -1e29, jnp.exp(mt - mstar), 0.0)
    sstar = jnp.sum(w * sp, axis=0, keepdims=True) + wt * st_ref[...]
    rstar = _col(wt) * rt_ref[...]
    for i in range(_NW):
        wcol = _col(w[i:i + 1])
        ri = r_ref[pl.ds(i * _B, _B), :]
        ri = jnp.where(wcol > 0.0, ri, 0.0)
        rstar = rstar + wcol * ri
    r_fin = rstar / (_col(sstar) + 1e-16)
    h = h_ref[...]
    q_star = jnp.concatenate([h, r_fin], axis=1)
    g = (lax.dot_general(q_star, wih_ref[...], (((1,), (1,)), ((), ())),
                         preferred_element_type=_F32)
         + lax.dot_general(h, whh_ref[...], (((1,), (1,)), ((), ())),
                           preferred_element_type=_F32)
         + bias_ref[...])
    gi = jax.nn.sigmoid(g[:, :_D])
    gf = jax.nn.sigmoid(g[:, _D:2 * _D])
    gg = jnp.tanh(g[:, 2 * _D:3 * _D])
    go = jax.nn.sigmoid(g[:, 3 * _D:])
    c_new = gf * c_ref[...] + gi * gg
    h_new = go * jnp.tanh(c_new)
    hn_ref[...] = h_new
    cn_ref[...] = c_new
    q_ref[...] = q_star


def _merge_lstm(mp, sp, rp, mt, st, rt, h, c, W_ih, W_hh, bias):
    return pl.pallas_call(
        _ml_body,
        out_shape=[
            jax.ShapeDtypeStruct((_B, _D), _F32),
            jax.ShapeDtypeStruct((_B, _D), _F32),
            jax.ShapeDtypeStruct((_B, 2 * _D), _F32),
        ],
    )(mp, sp, rp, mt, st, rt, h, c, W_ih, W_hh, bias)


def _swap(v, k):
    iota = lax.broadcasted_iota(jnp.int32, (_L,), 0)
    return v.at[jnp.bitwise_xor(iota, k)].get(mode="promise_in_bounds")


def _lanesum(v):
    for k in (8, 4, 2, 1):
        v = v + _swap(v, k)
    return v


def _lanemax(v):
    for k in (8, 4, 2, 1):
        v = jnp.maximum(v, _swap(v, k))
    return v


def _tree16(vs):
    cur = list(vs)
    for k in (8, 4, 2, 1):
        n = len(cur) // 2
        iota = lax.broadcasted_iota(jnp.int32, (_L,), 0)
        sel = (iota & k) == 0
        cur = [jnp.where(sel, cur[i] + _swap(cur[i], k),
                         cur[i + n] + _swap(cur[i + n], k))
               for i in range(n)]
    return cur[0]


def _splat(v, lane):
    return v.at[jnp.full((_L,), lane, jnp.int32)].get(
        mode="promise_in_bounds")


def _sc_attn(x, bat, h):
    mesh = plsc.VectorSubcoreMesh(core_axis_name="c", subcore_axis_name="s")

    @functools.partial(
        pl.kernel,
        mesh=mesh,
        out_type=[
            jax.ShapeDtypeStruct((_NW, _B), _F32),
            jax.ShapeDtypeStruct((_NW, _B), _F32),
            jax.ShapeDtypeStruct((_NW * _B + 8, _D), _F32),
        ],
        scratch_types=[
            pltpu.VMEM((_CH, _D), _F32),
            pltpu.VMEM((_CH + _L,), jnp.int32),
            pltpu.VMEM((_L, _D), _F32),
            pltpu.VMEM((1, _D), _F32),
            pltpu.VMEM((_L,), _F32),
            pltpu.VMEM((_L,), _F32),
            pltpu.VMEM((_B,), _F32),
            pltpu.VMEM((_B,), _F32),
            pltpu.VMEM((_L,), jnp.int32),
            pltpu.SemaphoreType.DMA,
        ],
    )
    def k(x_hbm, bat_hbm, h_hbm, m_out, s_out, r_out,
          xbuf, bbuf, hcur, rcur, mvec, svec, mloc, sloc, idxb, sem):
        cid = lax.axis_index("c")
        sid = lax.axis_index("s")
        wid = sid * 2 + cid
        start = wid * _C
        iota = lax.broadcasted_iota(jnp.int32, (_L,), 0)

        for j in range(_B // _L):
            mloc[pl.ds(j * _L, _L)] = jnp.full((_L,), _NEG, _F32)
            sloc[pl.ds(j * _L, _L)] = jnp.zeros((_L,), _F32)
        for j in range(_D // _L):
            rcur[0, pl.ds(j * _L, _L)] = jnp.zeros((_L,), _F32)
        mvec[...] = jnp.full((_L,), _NEG, _F32)
        svec[...] = jnp.zeros((_L,), _F32)

        def load_h(b):
            bsafe = jnp.minimum(b, _B - 1)
            pltpu.async_copy(h_hbm.at[jnp.full((_L,), bsafe, jnp.int32)],
                             hcur, sem).wait()

        def flush(b_old):
            m_old = mvec[...]
            s_old = svec[...]
            for j in range(_B // _L):
                sl = pl.ds(j * _L, _L)
                sel = (iota + j * _L) == b_old
                mloc[sl] = jnp.where(sel, m_old, mloc[sl])
                sloc[sl] = jnp.where(sel, s_old, sloc[sl])
            ridx = jnp.where(b_old < _B, wid * _B + b_old, _NW * _B)
            idxb[pl.ds(0, _L)] = jnp.where(iota == 0, ridx,
                                           idxb[pl.ds(0, _L)])
            pltpu.async_copy(rcur, r_out.at[idxb.at[pl.ds(0, 1)]],
                             sem).wait()

        def fast_group(gbase):
            accs = []
            for r in range(_L):
                acc = jnp.zeros((_L,), _F32)
                for j in range(_D // _L):
                    sl = pl.ds(j * _L, _L)
                    acc = acc + xbuf[gbase + r, sl] * hcur[0, sl]
                accs.append(acc)
            e_vec = _tree16(accs)
            m_grp = _lanemax(e_vec)
            p_vec = jnp.exp(e_vec - m_grp)
            s_grp = _lanesum(p_vec)
            m_v = mvec[...]
            m_new = jnp.maximum(m_v, m_grp)
            alpha = jnp.exp(m_v - m_new)
            beta = jnp.exp(m_grp - m_new)
            mvec[...] = m_new
            svec[...] = svec[...] * alpha + s_grp * beta
            ps = [_splat(p_vec, r) for r in range(_L)]
            for j in range(_D // _L):
                sl = pl.ds(j * _L, _L)
                t = jnp.zeros((_L,), _F32)
                for r in range(_L):
                    t = t + ps[r] * xbuf[gbase + r, sl]
                rcur[0, sl] = rcur[0, sl] * alpha + beta * t

        def slow_group(gbase, bv, b_cur):
            for r in range(_L):
                bi = bv[r]
                ch = bi != b_cur

                def on_new(b):
                    flush(b)
                    load_h(bi)
                    return bi

                b_cur = lax.cond(ch, on_new, lambda b: b, b_cur)
                chf = jnp.where(ch, 1.0, 0.0)
                m_old = mvec[...]
                m_v = m_old + chf * (_NEG - m_old)
                acc = jnp.zeros((_L,), _F32)
                for j in range(_D // _L):
                    sl = pl.ds(j * _L, _L)
                    acc = acc + xbuf[gbase + r, sl] * hcur[0, sl]
                e_v = _lanesum(acc)
                m_new = jnp.maximum(m_v, e_v)
                alpha = jnp.exp(m_v - m_new)
                p = jnp.exp(e_v - m_new)
                mvec[...] = m_new
                svec[...] = svec[...] * alpha + p
                for j in range(_D // _L):
                    sl = pl.ds(j * _L, _L)
                    rcur[0, sl] = rcur[0, sl] * alpha + p * xbuf[gbase + r, sl]
            return b_cur

        def group_body(g, b_cur):
            gbase = g * _L
            bv = bbuf[pl.ds(gbase, _L)]
            b_first = bv[0]
            b_last = bv[_L - 1]
            fast = jnp.logical_and(b_first == b_cur, b_last == b_cur)

            def do_fast(b):
                fast_group(gbase)
                return b

            def do_slow(b):
                return slow_group(gbase, bv, b)

            return lax.cond(fast, do_fast, do_slow, b_cur)

        def chunk_body(kk, b_cur):
            base = pl.multiple_of(start + kk * _CH, _CH)
            pltpu.sync_copy(bat_hbm.at[pl.ds(base, _CH)],
                            bbuf.at[pl.ds(0, _CH)])
            pltpu.sync_copy(x_hbm.at[pl.ds(base, _CH)], xbuf)
            return lax.fori_loop(0, _NG, group_body, b_cur)

        pltpu.sync_copy(bat_hbm.at[pl.ds(start, _CH)], bbuf.at[pl.ds(0, _CH)])
        b0 = bbuf[pl.ds(0, _L)][0]
        load_h(b0)
        b_cur = lax.fori_loop(0, _NCH, chunk_body, b0)

        flush(b_cur)
        pltpu.sync_copy(mloc, m_out.at[wid])
        pltpu.sync_copy(sloc, s_out.at[wid])

    return k(x, bat, h)


def kernel(x, batch, W_ih, W_hh, b_ih, b_hh):
    xsc = x[_NTC:]
    batsc = batch[_NTC:]
    xtc = jnp.pad(x[:_NTC], ((0, _NPTC - _NTC), (0, 0)))
    bat3 = jnp.pad(batch[:_NTC], (0, _NPTC - _NTC),
                   constant_values=_B).reshape(_NBLK, _R, 1)
    bias = (b_ih + b_hh).reshape(1, 4 * _D)
    h = jnp.zeros((_B, _D), _F32)
    c = jnp.zeros((_B, _D), _F32)
    mp = jnp.full((_NW, _B), _NEG, _F32)
    sp = jnp.zeros((_NW, _B), _F32)
    rp = jnp.zeros((_NW * _B + 8, _D), _F32)
    mt = jnp.full((1, _B), _NEG, _F32)
    st = jnp.zeros((1, _B), _F32)
    rt = jnp.zeros((_B, _D), _F32)
    q_star = None
    for _ in range(3):
        h, c, q_star = _merge_lstm(mp, sp, rp[:_NW * _B], mt, st, rt,
                                   h, c, W_ih, W_hh, bias)
        mp, sp, rp = _sc_attn(xsc, batsc, h)
        mt, st, rt = _tc_attn(bat3, xtc, h)
    _, _, q_star = _merge_lstm(mp, sp, rp[:_NW * _B], mt, st, rt,
                               h, c, W_ih, W_hh, bias)
    return q_star


# speedup vs baseline: 3.1762x; 1.0109x over previous
import functools

import jax
import jax.numpy as jnp
from jax import lax
from jax.experimental import pallas as pl
from jax.experimental.pallas import tpu as pltpu
from jax.experimental.pallas import tpu_sc as plsc

_N = 50000
_D = 256
_B = 256

_NW = 32
_C = 448
_CH = 448
_NCH = _C // _CH
_L = 16
_NG = _CH // _L
_NSC = _NW * _C
_NTC = _N - _NSC
_R = 2048
_NBLK = (_NTC + _R - 1) // _R
_NPTC = _NBLK * _R

_F32 = jnp.float32
_NEG = -1e30


def _col(v):
    ib = (lax.broadcasted_iota(jnp.int32, (_B, _B), 0) ==
          lax.broadcasted_iota(jnp.int32, (_B, _B), 1))
    return jnp.sum(jnp.where(ib, v, 0.0), axis=1, keepdims=True)


def _tca_body(bat_ref, x_ref, h_ref, m_out, s_out, r_out, m_s, s_s, r_s):
    blk = pl.program_id(0)

    @pl.when(blk == 0)
    def _init():
        m_s[...] = jnp.full_like(m_s, _NEG)
        s_s[...] = jnp.zeros_like(s_s)
        r_s[...] = jnp.zeros_like(r_s)

    bat = bat_ref[0]
    iota_b = lax.broadcasted_iota(jnp.int32, (_R, _B), 1)
    pmask = bat == iota_b
    pf = pmask.astype(_F32)
    h = h_ref[...]
    h_hi = h.astype(jnp.bfloat16).astype(_F32)
    h_lo = h - h_hi
    qg = (lax.dot_general(pf, h_hi, (((1,), (0,)), ((), ())),
                          preferred_element_type=_F32)
          + lax.dot_general(pf, h_lo, (((1,), (0,)), ((), ())),
                            preferred_element_type=_F32))
    xb = x_ref[...]
    e = jnp.sum(xb * qg, axis=1, keepdims=True)
    em = jnp.where(pmask, e, _NEG)
    mblk = jnp.max(em, axis=0, keepdims=True)
    m_old = m_s[...]
    m_new = jnp.maximum(m_old, mblk)
    scale = jnp.exp(m_old - m_new)
    gm = jnp.sum(jnp.where(pmask, m_new, 0.0), axis=1, keepdims=True)
    ex = jnp.exp(e - gm)
    pw = pf * ex
    sblk = jnp.sum(pw, axis=0, keepdims=True)
    m_s[...] = m_new
    s_s[...] = s_s[...] * scale + sblk
    scale_col = _col(scale)
    racc = lax.dot_general(pw, xb, (((0,), (0,)), ((), ())),
                           preferred_element_type=_F32)
    r_s[...] = r_s[...] * scale_col + racc

    @pl.when(blk == _NBLK - 1)
    def _fin():
        m_out[...] = m_s[...]
        s_out[...] = s_s[...]
        r_out[...] = r_s[...]


def _tc_attn(bat3, xtc, h):
    return pl.pallas_call(
        _tca_body,
        grid=(_NBLK,),
        in_specs=[
            pl.BlockSpec((1, _R, 1), lambda b: (b, 0, 0)),
            pl.BlockSpec((_R, _D), lambda b: (b, 0)),
            pl.BlockSpec((_B, _D), lambda b: (0, 0)),
        ],
        out_specs=[
            pl.BlockSpec((1, _B), lambda b: (0, 0)),
            pl.BlockSpec((1, _B), lambda b: (0, 0)),
            pl.BlockSpec((_B, _D), lambda b: (0, 0)),
        ],
        out_shape=[
            jax.ShapeDtypeStruct((1, _B), _F32),
            jax.ShapeDtypeStruct((1, _B), _F32),
            jax.ShapeDtypeStruct((_B, _D), _F32),
        ],
        scratch_shapes=[
            pltpu.VMEM((1, _B), _F32),
            pltpu.VMEM((1, _B), _F32),
            pltpu.VMEM((_B, _D), _F32),
        ],
    )(bat3, xtc, h)


def _ml_body(m_ref, s_ref, r_ref, mt_ref, st_ref, rt_ref, h_ref, c_ref,
             wih_ref, whh_ref, bias_ref, hn_ref, cn_ref, q_ref):
    mp = m_ref[...]
    sp = s_ref[...]
    mt = mt_ref[...]
    mstar = jnp.maximum(jnp.max(mp, axis=0, keepdims=True), mt)
    valid = mp > -1e29
    w = jnp.where(valid, jnp.exp(mp - mstar), 0.0)
    wt = jnp.where(mt > -1e29, jnp.exp(mt - mstar), 0.0)
    sstar = jnp.sum(w * sp, axis=0, keepdims=True) + wt * st_ref[...]
    rstar = _col(wt) * rt_ref[...]
    for i in range(_NW):
        wcol = _col(w[i:i + 1])
        ri = r_ref[pl.ds(i * _B, _B), :]
        ri = jnp.where(wcol > 0.0, ri, 0.0)
        rstar = rstar + wcol * ri
    r_fin = rstar / (_col(sstar) + 1e-16)
    h = h_ref[...]
    q_star = jnp.concatenate([h, r_fin], axis=1)
    g = (lax.dot_general(q_star, wih_ref[...], (((1,), (1,)), ((), ())),
                         preferred_element_type=_F32)
         + lax.dot_general(h, whh_ref[...], (((1,), (1,)), ((), ())),
                           preferred_element_type=_F32)
         + bias_ref[...])
    gi = jax.nn.sigmoid(g[:, :_D])
    gf = jax.nn.sigmoid(g[:, _D:2 * _D])
    gg = jnp.tanh(g[:, 2 * _D:3 * _D])
    go = jax.nn.sigmoid(g[:, 3 * _D:])
    c_new = gf * c_ref[...] + gi * gg
    h_new = go * jnp.tanh(c_new)
    hn_ref[...] = h_new
    cn_ref[...] = c_new
    q_ref[...] = q_star


def _merge_lstm(mp, sp, rp, mt, st, rt, h, c, W_ih, W_hh, bias):
    return pl.pallas_call(
        _ml_body,
        out_shape=[
            jax.ShapeDtypeStruct((_B, _D), _F32),
            jax.ShapeDtypeStruct((_B, _D), _F32),
            jax.ShapeDtypeStruct((_B, 2 * _D), _F32),
        ],
    )(mp, sp, rp, mt, st, rt, h, c, W_ih, W_hh, bias)


def _swap(v, k):
    iota = lax.broadcasted_iota(jnp.int32, (_L,), 0)
    return v.at[jnp.bitwise_xor(iota, k)].get(mode="promise_in_bounds")


def _lanesum(v):
    for k in (8, 4, 2, 1):
        v = v + _swap(v, k)
    return v


def _lanemax(v):
    for k in (8, 4, 2, 1):
        v = jnp.maximum(v, _swap(v, k))
    return v


def _tree16(vs):
    cur = list(vs)
    for k in (8, 4, 2, 1):
        n = len(cur) // 2
        iota = lax.broadcasted_iota(jnp.int32, (_L,), 0)
        sel = (iota & k) == 0
        cur = [jnp.where(sel, cur[i] + _swap(cur[i], k),
                         cur[i + n] + _swap(cur[i + n], k))
               for i in range(n)]
    return cur[0]


def _splat(v, lane):
    return v.at[jnp.full((_L,), lane, jnp.int32)].get(
        mode="promise_in_bounds")


def _sc_attn(x, bat, h):
    mesh = plsc.VectorSubcoreMesh(core_axis_name="c", subcore_axis_name="s")

    @functools.partial(
        pl.kernel,
        mesh=mesh,
        out_type=[
            jax.ShapeDtypeStruct((_NW, _B), _F32),
            jax.ShapeDtypeStruct((_NW, _B), _F32),
            jax.ShapeDtypeStruct((_NW * _B + 8, _D), _F32),
        ],
        scratch_types=[
            pltpu.VMEM((_CH, _D), _F32),
            pltpu.VMEM((_CH + _L,), jnp.int32),
            pltpu.VMEM((_L, _D), _F32),
            pltpu.VMEM((1, _D), _F32),
            pltpu.VMEM((_L,), _F32),
            pltpu.VMEM((_L,), _F32),
            pltpu.VMEM((_B,), _F32),
            pltpu.VMEM((_B,), _F32),
            pltpu.VMEM((_L,), jnp.int32),
            pltpu.SemaphoreType.DMA,
        ],
    )
    def k(x_hbm, bat_hbm, h_hbm, m_out, s_out, r_out,
          xbuf, bbuf, hcur, rcur, mvec, svec, mloc, sloc, idxb, sem):
        cid = lax.axis_index("c")
        sid = lax.axis_index("s")
        wid = sid * 2 + cid
        start = wid * _C
        iota = lax.broadcasted_iota(jnp.int32, (_L,), 0)

        for j in range(_B // _L):
            mloc[pl.ds(j * _L, _L)] = jnp.full((_L,), _NEG, _F32)
            sloc[pl.ds(j * _L, _L)] = jnp.zeros((_L,), _F32)
        for j in range(_D // _L):
            rcur[0, pl.ds(j * _L, _L)] = jnp.zeros((_L,), _F32)
        mvec[...] = jnp.full((_L,), _NEG, _F32)
        svec[...] = jnp.zeros((_L,), _F32)

        def load_h(b):
            bsafe = jnp.minimum(b, _B - 1)
            pltpu.async_copy(h_hbm.at[jnp.full((_L,), bsafe, jnp.int32)],
                             hcur, sem).wait()

        def flush(b_old):
            m_old = mvec[...]
            s_old = svec[...]
            for j in range(_B // _L):
                sl = pl.ds(j * _L, _L)
                sel = (iota + j * _L) == b_old
                mloc[sl] = jnp.where(sel, m_old, mloc[sl])
                sloc[sl] = jnp.where(sel, s_old, sloc[sl])
            ridx = jnp.where(b_old < _B, wid * _B + b_old, _NW * _B)
            idxb[pl.ds(0, _L)] = jnp.where(iota == 0, ridx,
                                           idxb[pl.ds(0, _L)])
            pltpu.async_copy(rcur, r_out.at[idxb.at[pl.ds(0, 1)]],
                             sem).wait()

        def fast_group(gbase):
            hs = [hcur[0, pl.ds(j * _L, _L)] for j in range(_D // _L)]
            accs = []
            for r in range(_L):
                acc = jnp.zeros((_L,), _F32)
                for j in range(_D // _L):
                    acc = acc + xbuf[gbase + r, pl.ds(j * _L, _L)] * hs[j]
                accs.append(acc)
            e_vec = _tree16(accs)
            m_grp = _lanemax(e_vec)
            p_vec = jnp.exp(e_vec - m_grp)
            s_grp = _lanesum(p_vec)
            m_v = mvec[...]
            m_new = jnp.maximum(m_v, m_grp)
            alpha = jnp.exp(m_v - m_new)
            beta = jnp.exp(m_grp - m_new)
            mvec[...] = m_new
            svec[...] = svec[...] * alpha + s_grp * beta
            ps = [_splat(p_vec, r) for r in range(_L)]
            for j in range(_D // _L):
                sl = pl.ds(j * _L, _L)
                t = jnp.zeros((_L,), _F32)
                for r in range(_L):
                    t = t + ps[r] * xbuf[gbase + r, sl]
                rcur[0, sl] = rcur[0, sl] * alpha + beta * t

        def slow_group(gbase, bv, b_cur):
            for r in range(_L):
                bi = bv[r]
                ch = bi != b_cur

                def on_new(b):
                    flush(b)
                    load_h(bi)
                    return bi

                b_cur = lax.cond(ch, on_new, lambda b: b, b_cur)
                chf = jnp.where(ch, 1.0, 0.0)
                m_old = mvec[...]
                m_v = m_old + chf * (_NEG - m_old)
                acc = jnp.zeros((_L,), _F32)
                for j in range(_D // _L):
                    sl = pl.ds(j * _L, _L)
                    acc = acc + xbuf[gbase + r, sl] * hcur[0, sl]
                e_v = _lanesum(acc)
                m_new = jnp.maximum(m_v, e_v)
                alpha = jnp.exp(m_v - m_new)
                p = jnp.exp(e_v - m_new)
                mvec[...] = m_new
                svec[...] = svec[...] * alpha + p
                for j in range(_D // _L):
                    sl = pl.ds(j * _L, _L)
                    rcur[0, sl] = rcur[0, sl] * alpha + p * xbuf[gbase + r, sl]
            return b_cur

        def group_body(g, b_cur):
            gbase = g * _L
            bv = bbuf[pl.ds(gbase, _L)]
            b_first = bv[0]
            b_last = bv[_L - 1]
            fast = jnp.logical_and(b_first == b_cur, b_last == b_cur)

            def do_fast(b):
                fast_group(gbase)
                return b

            def do_slow(b):
                return slow_group(gbase, bv, b)

            return lax.cond(fast, do_fast, do_slow, b_cur)

        def chunk_body(kk, b_cur):
            base = pl.multiple_of(start + kk * _CH, _CH)
            pltpu.sync_copy(bat_hbm.at[pl.ds(base, _CH)],
                            bbuf.at[pl.ds(0, _CH)])
            pltpu.sync_copy(x_hbm.at[pl.ds(base, _CH)], xbuf)
            return lax.fori_loop(0, _NG, group_body, b_cur)

        pltpu.sync_copy(bat_hbm.at[pl.ds(start, _CH)], bbuf.at[pl.ds(0, _CH)])
        b0 = bbuf[pl.ds(0, _L)][0]
        load_h(b0)
        b_cur = lax.fori_loop(0, _NCH, chunk_body, b0)

        flush(b_cur)
        pltpu.sync_copy(mloc, m_out.at[wid])
        pltpu.sync_copy(sloc, s_out.at[wid])

    return k(x, bat, h)


def kernel(x, batch, W_ih, W_hh, b_ih, b_hh):
    xsc = x[_NTC:]
    batsc = batch[_NTC:]
    xtc = jnp.pad(x[:_NTC], ((0, _NPTC - _NTC), (0, 0)))
    bat3 = jnp.pad(batch[:_NTC], (0, _NPTC - _NTC),
                   constant_values=_B).reshape(_NBLK, _R, 1)
    bias = (b_ih + b_hh).reshape(1, 4 * _D)
    h = jnp.zeros((_B, _D), _F32)
    c = jnp.zeros((_B, _D), _F32)
    mp = jnp.full((_NW, _B), _NEG, _F32)
    sp = jnp.zeros((_NW, _B), _F32)
    rp = jnp.zeros((_NW * _B + 8, _D), _F32)
    mt = jnp.full((1, _B), _NEG, _F32)
    st = jnp.zeros((1, _B), _F32)
    rt = jnp.zeros((_B, _D), _F32)
    q_star = None
    for _ in range(3):
        h, c, q_star = _merge_lstm(mp, sp, rp[:_NW * _B], mt, st, rt,
                                   h, c, W_ih, W_hh, bias)
        mp, sp, rp = _sc_attn(xsc, batsc, h)
        mt, st, rt = _tc_attn(bat3, xtc, h)
    _, _, q_star = _merge_lstm(mp, sp, rp[:_NW * _B], mt, st, rt,
                               h, c, W_ih, W_hh, bias)
    return q_star


# speedup vs baseline: 3.2213x; 1.0142x over previous
import functools

import jax
import jax.numpy as jnp
from jax import lax
from jax.experimental import pallas as pl
from jax.experimental.pallas import tpu as pltpu
from jax.experimental.pallas import tpu_sc as plsc

_N = 50000
_D = 256
_B = 256

_NW = 32
_C = 448
_CH = 448
_NCH = _C // _CH
_L = 16
_NG = _CH // _L
_NSC = _NW * _C
_NTC = _N - _NSC
_R = 2048
_NBLK = (_NTC + _R - 1) // _R
_NPTC = _NBLK * _R

_F32 = jnp.float32
_NEG = -1e30


def _col(v):
    ib = (lax.broadcasted_iota(jnp.int32, (_B, _B), 0) ==
          lax.broadcasted_iota(jnp.int32, (_B, _B), 1))
    return jnp.sum(jnp.where(ib, v, 0.0), axis=1, keepdims=True)


def _tca_body(bat_ref, x_ref, h_ref, m_out, s_out, r_out, m_s, s_s, r_s):
    blk = pl.program_id(0)

    @pl.when(blk == 0)
    def _init():
        m_s[...] = jnp.full_like(m_s, _NEG)
        s_s[...] = jnp.zeros_like(s_s)
        r_s[...] = jnp.zeros_like(r_s)

    bat = bat_ref[0]
    iota_b = lax.broadcasted_iota(jnp.int32, (_R, _B), 1)
    pmask = bat == iota_b
    pf = pmask.astype(_F32)
    h = h_ref[...]
    h_hi = h.astype(jnp.bfloat16).astype(_F32)
    h_lo = h - h_hi
    qg = (lax.dot_general(pf, h_hi, (((1,), (0,)), ((), ())),
                          preferred_element_type=_F32)
          + lax.dot_general(pf, h_lo, (((1,), (0,)), ((), ())),
                            preferred_element_type=_F32))
    xb = x_ref[...]
    e = jnp.sum(xb * qg, axis=1, keepdims=True)
    em = jnp.where(pmask, e, _NEG)
    mblk = jnp.max(em, axis=0, keepdims=True)
    m_old = m_s[...]
    m_new = jnp.maximum(m_old, mblk)
    scale = jnp.exp(m_old - m_new)
    gm = jnp.sum(jnp.where(pmask, m_new, 0.0), axis=1, keepdims=True)
    ex = jnp.exp(e - gm)
    pw = pf * ex
    sblk = jnp.sum(pw, axis=0, keepdims=True)
    m_s[...] = m_new
    s_s[...] = s_s[...] * scale + sblk
    scale_col = _col(scale)
    racc = lax.dot_general(pw, xb, (((0,), (0,)), ((), ())),
                           preferred_element_type=_F32)
    r_s[...] = r_s[...] * scale_col + racc

    @pl.when(blk == _NBLK - 1)
    def _fin():
        m_out[...] = m_s[...]
        s_out[...] = s_s[...]
        r_out[...] = r_s[...]


def _tc_attn(bat3, xtc, h):
    return pl.pallas_call(
        _tca_body,
        grid=(_NBLK,),
        in_specs=[
            pl.BlockSpec((1, _R, 1), lambda b: (b, 0, 0)),
            pl.BlockSpec((_R, _D), lambda b: (b, 0)),
            pl.BlockSpec((_B, _D), lambda b: (0, 0)),
        ],
        out_specs=[
            pl.BlockSpec((1, _B), lambda b: (0, 0)),
            pl.BlockSpec((1, _B), lambda b: (0, 0)),
            pl.BlockSpec((_B, _D), lambda b: (0, 0)),
        ],
        out_shape=[
            jax.ShapeDtypeStruct((1, _B), _F32),
            jax.ShapeDtypeStruct((1, _B), _F32),
            jax.ShapeDtypeStruct((_B, _D), _F32),
        ],
        scratch_shapes=[
            pltpu.VMEM((1, _B), _F32),
            pltpu.VMEM((1, _B), _F32),
            pltpu.VMEM((_B, _D), _F32),
        ],
    )(bat3, xtc, h)


def _ml_body(m_ref, s_ref, r_ref, mt_ref, st_ref, rt_ref, h_ref, c_ref,
             wih_ref, whh_ref, bias_ref, hn_ref, cn_ref, q_ref):
    mp = m_ref[...]
    sp = s_ref[...]
    mt = mt_ref[...]
    mstar = jnp.maximum(jnp.max(mp, axis=0, keepdims=True), mt)
    valid = mp > -1e29
    w = jnp.where(valid, jnp.exp(mp - mstar), 0.0)
    wt = jnp.where(mt > -1e29, jnp.exp(mt - mstar), 0.0)
    sstar = jnp.sum(w * sp, axis=0, keepdims=True) + wt * st_ref[...]
    rstar = _col(wt) * rt_ref[...]
    for i in range(_NW):
        wcol = _col(w[i:i + 1])
        ri = r_ref[pl.ds(i * _B, _B), :]
        ri = jnp.where(wcol > 0.0, ri, 0.0)
        rstar = rstar + wcol * ri
    r_fin = rstar / (_col(sstar) + 1e-16)
    h = h_ref[...]
    q_star = jnp.concatenate([h, r_fin], axis=1)
    g = (lax.dot_general(q_star, wih_ref[...], (((1,), (1,)), ((), ())),
                         preferred_element_type=_F32)
         + lax.dot_general(h, whh_ref[...], (((1,), (1,)), ((), ())),
                           preferred_element_type=_F32)
         + bias_ref[...])
    gi = jax.nn.sigmoid(g[:, :_D])
    gf = jax.nn.sigmoid(g[:, _D:2 * _D])
    gg = jnp.tanh(g[:, 2 * _D:3 * _D])
    go = jax.nn.sigmoid(g[:, 3 * _D:])
    c_new = gf * c_ref[...] + gi * gg
    h_new = go * jnp.tanh(c_new)
    hn_ref[...] = h_new
    cn_ref[...] = c_new
    q_ref[...] = q_star


def _merge_lstm(mp, sp, rp, mt, st, rt, h, c, W_ih, W_hh, bias):
    return pl.pallas_call(
        _ml_body,
        out_shape=[
            jax.ShapeDtypeStruct((_B, _D), _F32),
            jax.ShapeDtypeStruct((_B, _D), _F32),
            jax.ShapeDtypeStruct((_B, 2 * _D), _F32),
        ],
    )(mp, sp, rp, mt, st, rt, h, c, W_ih, W_hh, bias)


def _swap(v, k):
    iota = lax.broadcasted_iota(jnp.int32, (_L,), 0)
    return v.at[jnp.bitwise_xor(iota, k)].get(mode="promise_in_bounds")


def _lanesum(v):
    for k in (8, 4, 2, 1):
        v = v + _swap(v, k)
    return v


def _lanemax(v):
    for k in (8, 4, 2, 1):
        v = jnp.maximum(v, _swap(v, k))
    return v


def _tree16(vs):
    cur = list(vs)
    for k in (8, 4, 2, 1):
        n = len(cur) // 2
        iota = lax.broadcasted_iota(jnp.int32, (_L,), 0)
        sel = (iota & k) == 0
        cur = [jnp.where(sel, cur[i] + _swap(cur[i], k),
                         cur[i + n] + _swap(cur[i + n], k))
               for i in range(n)]
    return cur[0]


def _splat(v, lane):
    return v.at[jnp.full((_L,), lane, jnp.int32)].get(
        mode="promise_in_bounds")


def _sc_attn(x, bat, h):
    mesh = plsc.VectorSubcoreMesh(core_axis_name="c", subcore_axis_name="s")

    @functools.partial(
        pl.kernel,
        mesh=mesh,
        out_type=[
            jax.ShapeDtypeStruct((_NW, _B), _F32),
            jax.ShapeDtypeStruct((_NW, _B), _F32),
            jax.ShapeDtypeStruct((_NW * _B + 8, _D), _F32),
        ],
        scratch_types=[
            pltpu.VMEM((_CH, _D), _F32),
            pltpu.VMEM((_CH + _L,), jnp.int32),
            pltpu.VMEM((_L, _D), _F32),
            pltpu.VMEM((1, _D), _F32),
            pltpu.VMEM((_L,), _F32),
            pltpu.VMEM((_L,), _F32),
            pltpu.VMEM((_B,), _F32),
            pltpu.VMEM((_B,), _F32),
            pltpu.VMEM((_L,), jnp.int32),
            pltpu.SemaphoreType.DMA,
        ],
    )
    def k(x_hbm, bat_hbm, h_hbm, m_out, s_out, r_out,
          xbuf, bbuf, hcur, rcur, mvec, svec, mloc, sloc, idxb, sem):
        cid = lax.axis_index("c")
        sid = lax.axis_index("s")
        wid = sid * 2 + cid
        start = wid * _C
        iota = lax.broadcasted_iota(jnp.int32, (_L,), 0)

        for j in range(_B // _L):
            mloc[pl.ds(j * _L, _L)] = jnp.full((_L,), _NEG, _F32)
            sloc[pl.ds(j * _L, _L)] = jnp.zeros((_L,), _F32)
        for j in range(_D // _L):
            rcur[0, pl.ds(j * _L, _L)] = jnp.zeros((_L,), _F32)
        mvec[...] = jnp.full((_L,), _NEG, _F32)
        svec[...] = jnp.zeros((_L,), _F32)

        def load_h(b):
            bsafe = jnp.minimum(b, _B - 1)
            pltpu.async_copy(h_hbm.at[jnp.full((_L,), bsafe, jnp.int32)],
                             hcur, sem).wait()

        def flush(b_old):
            m_old = mvec[...]
            s_old = svec[...]
            for j in range(_B // _L):
                sl = pl.ds(j * _L, _L)
                sel = (iota + j * _L) == b_old
                mloc[sl] = jnp.where(sel, m_old, mloc[sl])
                sloc[sl] = jnp.where(sel, s_old, sloc[sl])
            ridx = jnp.where(b_old < _B, wid * _B + b_old, _NW * _B)
            idxb[pl.ds(0, _L)] = jnp.where(iota == 0, ridx,
                                           idxb[pl.ds(0, _L)])
            pltpu.async_copy(rcur, r_out.at[idxb.at[pl.ds(0, 1)]],
                             sem).wait()

        def fast_group(gbase):
            hs = [hcur[0, pl.ds(j * _L, _L)] for j in range(_D // _L)]
            accs = [jnp.zeros((_L,), _F32) for _ in range(_L)]
            for j in range(_D // _L):
                sl = pl.ds(j * _L, _L)
                for r in range(_L):
                    accs[r] = accs[r] + xbuf[gbase + r, sl] * hs[j]
            e_vec = _tree16(accs)
            m_grp = _lanemax(e_vec)
            p_vec = jnp.exp(e_vec - m_grp)
            s_grp = _lanesum(p_vec)
            m_v = mvec[...]
            m_new = jnp.maximum(m_v, m_grp)
            alpha = jnp.exp(m_v - m_new)
            beta = jnp.exp(m_grp - m_new)
            mvec[...] = m_new
            svec[...] = svec[...] * alpha + s_grp * beta
            ps = [_splat(p_vec, r) for r in range(_L)]
            ts = [jnp.zeros((_L,), _F32) for _ in range(_D // _L)]
            for r in range(_L):
                for j in range(_D // _L):
                    ts[j] = ts[j] + ps[r] * xbuf[gbase + r, pl.ds(j * _L, _L)]
            for j in range(_D // _L):
                sl = pl.ds(j * _L, _L)
                rcur[0, sl] = rcur[0, sl] * alpha + beta * ts[j]

        def slow_group(gbase, bv, b_cur):
            for r in range(_L):
                bi = bv[r]
                ch = bi != b_cur

                def on_new(b):
                    flush(b)
                    load_h(bi)
                    return bi

                b_cur = lax.cond(ch, on_new, lambda b: b, b_cur)
                chf = jnp.where(ch, 1.0, 0.0)
                m_old = mvec[...]
                m_v = m_old + chf * (_NEG - m_old)
                acc = jnp.zeros((_L,), _F32)
                for j in range(_D // _L):
                    sl = pl.ds(j * _L, _L)
                    acc = acc + xbuf[gbase + r, sl] * hcur[0, sl]
                e_v = _lanesum(acc)
                m_new = jnp.maximum(m_v, e_v)
                alpha = jnp.exp(m_v - m_new)
                p = jnp.exp(e_v - m_new)
                mvec[...] = m_new
                svec[...] = svec[...] * alpha + p
                for j in range(_D // _L):
                    sl = pl.ds(j * _L, _L)
                    rcur[0, sl] = rcur[0, sl] * alpha + p * xbuf[gbase + r, sl]
            return b_cur

        def group_body(g, b_cur):
            gbase = g * _L
            bv = bbuf[pl.ds(gbase, _L)]
            b_first = bv[0]
            b_last = bv[_L - 1]
            fast = jnp.logical_and(b_first == b_cur, b_last == b_cur)

            def do_fast(b):
                fast_group(gbase)
                return b

            def do_slow(b):
                return slow_group(gbase, bv, b)

            return lax.cond(fast, do_fast, do_slow, b_cur)

        def chunk_body(kk, b_cur):
            base = pl.multiple_of(start + kk * _CH, _CH)
            pltpu.sync_copy(bat_hbm.at[pl.ds(base, _CH)],
                            bbuf.at[pl.ds(0, _CH)])
            pltpu.sync_copy(x_hbm.at[pl.ds(base, _CH)], xbuf)
            return lax.fori_loop(0, _NG, group_body, b_cur)

        pltpu.sync_copy(bat_hbm.at[pl.ds(start, _CH)], bbuf.at[pl.ds(0, _CH)])
        b0 = bbuf[pl.ds(0, _L)][0]
        load_h(b0)
        b_cur = lax.fori_loop(0, _NCH, chunk_body, b0)

        flush(b_cur)
        pltpu.sync_copy(mloc, m_out.at[wid])
        pltpu.sync_copy(sloc, s_out.at[wid])

    return k(x, bat, h)


def kernel(x, batch, W_ih, W_hh, b_ih, b_hh):
    xsc = x[_NTC:]
    batsc = batch[_NTC:]
    xtc = jnp.pad(x[:_NTC], ((0, _NPTC - _NTC), (0, 0)))
    bat3 = jnp.pad(batch[:_NTC], (0, _NPTC - _NTC),
                   constant_values=_B).reshape(_NBLK, _R, 1)
    bias = (b_ih + b_hh).reshape(1, 4 * _D)
    h = jnp.zeros((_B, _D), _F32)
    c = jnp.zeros((_B, _D), _F32)
    mp = jnp.full((_NW, _B), _NEG, _F32)
    sp = jnp.zeros((_NW, _B), _F32)
    rp = jnp.zeros((_NW * _B + 8, _D), _F32)
    mt = jnp.full((1, _B), _NEG, _F32)
    st = jnp.zeros((1, _B), _F32)
    rt = jnp.zeros((_B, _D), _F32)
    q_star = None
    for _ in range(3):
        h, c, q_star = _merge_lstm(mp, sp, rp[:_NW * _B], mt, st, rt,
                                   h, c, W_ih, W_hh, bias)
        mp, sp, rp = _sc_attn(xsc, batsc, h)
        mt, st, rt = _tc_attn(bat3, xtc, h)
    _, _, q_star = _merge_lstm(mp, sp, rp[:_NW * _B], mt, st, rt,
                               h, c, W_ih, W_hh, bias)
    return q_star


# speedup vs baseline: 3.2820x; 1.0188x over previous
import functools

import jax
import jax.numpy as jnp
from jax import lax
from jax.experimental import pallas as pl
from jax.experimental.pallas import tpu as pltpu
from jax.experimental.pallas import tpu_sc as plsc

_N = 50000
_D = 256
_B = 256

_NW = 32
_C = 320
_CH = 320
_NCH = _C // _CH
_L = 16
_NG = _CH // _L
_NSC = _NW * _C
_NTC = _N - _NSC
_R = 2048
_NBLK = (_NTC + _R - 1) // _R
_NPTC = _NBLK * _R

_F32 = jnp.float32
_NEG = -1e30


def _col(v):
    ib = (lax.broadcasted_iota(jnp.int32, (_B, _B), 0) ==
          lax.broadcasted_iota(jnp.int32, (_B, _B), 1))
    return jnp.sum(jnp.where(ib, v, 0.0), axis=1, keepdims=True)


def _tca_body(bat_ref, x_ref, h_ref, m_out, s_out, r_out, m_s, s_s, r_s):
    blk = pl.program_id(0)

    @pl.when(blk == 0)
    def _init():
        m_s[...] = jnp.full_like(m_s, _NEG)
        s_s[...] = jnp.zeros_like(s_s)
        r_s[...] = jnp.zeros_like(r_s)

    bat = bat_ref[0]
    iota_b = lax.broadcasted_iota(jnp.int32, (_R, _B), 1)
    pmask = bat == iota_b
    pf = pmask.astype(_F32)
    h = h_ref[...]
    h_hi = h.astype(jnp.bfloat16).astype(_F32)
    h_lo = h - h_hi
    qg = (lax.dot_general(pf, h_hi, (((1,), (0,)), ((), ())),
                          preferred_element_type=_F32)
          + lax.dot_general(pf, h_lo, (((1,), (0,)), ((), ())),
                            preferred_element_type=_F32))
    xb = x_ref[...]
    e = jnp.sum(xb * qg, axis=1, keepdims=True)
    em = jnp.where(pmask, e, _NEG)
    mblk = jnp.max(em, axis=0, keepdims=True)
    m_old = m_s[...]
    m_new = jnp.maximum(m_old, mblk)
    scale = jnp.exp(m_old - m_new)
    gm = jnp.sum(jnp.where(pmask, m_new, 0.0), axis=1, keepdims=True)
    ex = jnp.exp(e - gm)
    pw = pf * ex
    sblk = jnp.sum(pw, axis=0, keepdims=True)
    m_s[...] = m_new
    s_s[...] = s_s[...] * scale + sblk
    scale_col = _col(scale)
    racc = lax.dot_general(pw, xb, (((0,), (0,)), ((), ())),
                           preferred_element_type=_F32)
    r_s[...] = r_s[...] * scale_col + racc

    @pl.when(blk == _NBLK - 1)
    def _fin():
        m_out[...] = m_s[...]
        s_out[...] = s_s[...]
        r_out[...] = r_s[...]


def _tc_attn(bat3, xtc, h):
    return pl.pallas_call(
        _tca_body,
        grid=(_NBLK,),
        in_specs=[
            pl.BlockSpec((1, _R, 1), lambda b: (b, 0, 0)),
            pl.BlockSpec((_R, _D), lambda b: (b, 0)),
            pl.BlockSpec((_B, _D), lambda b: (0, 0)),
        ],
        out_specs=[
            pl.BlockSpec((1, _B), lambda b: (0, 0)),
            pl.BlockSpec((1, _B), lambda b: (0, 0)),
            pl.BlockSpec((_B, _D), lambda b: (0, 0)),
        ],
        out_shape=[
            jax.ShapeDtypeStruct((1, _B), _F32),
            jax.ShapeDtypeStruct((1, _B), _F32),
            jax.ShapeDtypeStruct((_B, _D), _F32),
        ],
        scratch_shapes=[
            pltpu.VMEM((1, _B), _F32),
            pltpu.VMEM((1, _B), _F32),
            pltpu.VMEM((_B, _D), _F32),
        ],
    )(bat3, xtc, h)


def _ml_body(m_ref, s_ref, r_ref, mt_ref, st_ref, rt_ref, h_ref, c_ref,
             wih_ref, whh_ref, bias_ref, hn_ref, cn_ref, q_ref):
    mp = m_ref[...]
    sp = s_ref[...]
    mt = mt_ref[...]
    mstar = jnp.maximum(jnp.max(mp, axis=0, keepdims=True), mt)
    valid = mp > -1e29
    w = jnp.where(valid, jnp.exp(mp - mstar), 0.0)
    wt = jnp.where(mt > -1e29, jnp.exp(mt - mstar), 0.0)
    sstar = jnp.sum(w * sp, axis=0, keepdims=True) + wt * st_ref[...]
    rstar = _col(wt) * rt_ref[...]
    for i in range(_NW):
        wcol = _col(w[i:i + 1])
        ri = r_ref[pl.ds(i * _B, _B), :]
        ri = jnp.where(wcol > 0.0, ri, 0.0)
        rstar = rstar + wcol * ri
    r_fin = rstar / (_col(sstar) + 1e-16)
    h = h_ref[...]
    q_star = jnp.concatenate([h, r_fin], axis=1)
    g = (lax.dot_general(q_star, wih_ref[...], (((1,), (1,)), ((), ())),
                         preferred_element_type=_F32)
         + lax.dot_general(h, whh_ref[...], (((1,), (1,)), ((), ())),
                           preferred_element_type=_F32)
         + bias_ref[...])
    gi = jax.nn.sigmoid(g[:, :_D])
    gf = jax.nn.sigmoid(g[:, _D:2 * _D])
    gg = jnp.tanh(g[:, 2 * _D:3 * _D])
    go = jax.nn.sigmoid(g[:, 3 * _D:])
    c_new = gf * c_ref[...] + gi * gg
    h_new = go * jnp.tanh(c_new)
    hn_ref[...] = h_new
    cn_ref[...] = c_new
    q_ref[...] = q_star


def _merge_lstm(mp, sp, rp, mt, st, rt, h, c, W_ih, W_hh, bias):
    return pl.pallas_call(
        _ml_body,
        out_shape=[
            jax.ShapeDtypeStruct((_B, _D), _F32),
            jax.ShapeDtypeStruct((_B, _D), _F32),
            jax.ShapeDtypeStruct((_B, 2 * _D), _F32),
        ],
    )(mp, sp, rp, mt, st, rt, h, c, W_ih, W_hh, bias)


def _swap(v, k):
    iota = lax.broadcasted_iota(jnp.int32, (_L,), 0)
    return v.at[jnp.bitwise_xor(iota, k)].get(mode="promise_in_bounds")


def _lanesum(v):
    for k in (8, 4, 2, 1):
        v = v + _swap(v, k)
    return v


def _lanemax(v):
    for k in (8, 4, 2, 1):
        v = jnp.maximum(v, _swap(v, k))
    return v


def _tree16(vs):
    cur = list(vs)
    for k in (8, 4, 2, 1):
        n = len(cur) // 2
        iota = lax.broadcasted_iota(jnp.int32, (_L,), 0)
        sel = (iota & k) == 0
        cur = [jnp.where(sel, cur[i] + _swap(cur[i], k),
                         cur[i + n] + _swap(cur[i + n], k))
               for i in range(n)]
    return cur[0]


def _splat(v, lane):
    return v.at[jnp.full((_L,), lane, jnp.int32)].get(
        mode="promise_in_bounds")


def _sc_attn(x, bat, h):
    mesh = plsc.VectorSubcoreMesh(core_axis_name="c", subcore_axis_name="s")

    @functools.partial(
        pl.kernel,
        mesh=mesh,
        out_type=[
            jax.ShapeDtypeStruct((_NW, _B), _F32),
            jax.ShapeDtypeStruct((_NW, _B), _F32),
            jax.ShapeDtypeStruct((_NW * _B + 8, _D), _F32),
        ],
        scratch_types=[
            pltpu.VMEM((_CH, _D), _F32),
            pltpu.VMEM((_CH + _L,), jnp.int32),
            pltpu.VMEM((_L, _D), _F32),
            pltpu.VMEM((1, _D), _F32),
            pltpu.VMEM((_L,), _F32),
            pltpu.VMEM((_L,), _F32),
            pltpu.VMEM((_B,), _F32),
            pltpu.VMEM((_B,), _F32),
            pltpu.VMEM((_L,), jnp.int32),
            pltpu.SemaphoreType.DMA,
        ],
    )
    def k(x_hbm, bat_hbm, h_hbm, m_out, s_out, r_out,
          xbuf, bbuf, hcur, rcur, mvec, svec, mloc, sloc, idxb, sem):
        cid = lax.axis_index("c")
        sid = lax.axis_index("s")
        wid = sid * 2 + cid
        start = wid * _C
        iota = lax.broadcasted_iota(jnp.int32, (_L,), 0)

        for j in range(_B // _L):
            mloc[pl.ds(j * _L, _L)] = jnp.full((_L,), _NEG, _F32)
            sloc[pl.ds(j * _L, _L)] = jnp.zeros((_L,), _F32)
        for j in range(_D // _L):
            rcur[0, pl.ds(j * _L, _L)] = jnp.zeros((_L,), _F32)
        mvec[...] = jnp.full((_L,), _NEG, _F32)
        svec[...] = jnp.zeros((_L,), _F32)

        def load_h(b):
            bsafe = jnp.minimum(b, _B - 1)
            pltpu.async_copy(h_hbm.at[jnp.full((_L,), bsafe, jnp.int32)],
                             hcur, sem).wait()

        def flush(b_old):
            m_old = mvec[...]
            s_old = svec[...]
            for j in range(_B // _L):
                sl = pl.ds(j * _L, _L)
                sel = (iota + j * _L) == b_old
                mloc[sl] = jnp.where(sel, m_old, mloc[sl])
                sloc[sl] = jnp.where(sel, s_old, sloc[sl])
            ridx = jnp.where(b_old < _B, wid * _B + b_old, _NW * _B)
            idxb[pl.ds(0, _L)] = jnp.where(iota == 0, ridx,
                                           idxb[pl.ds(0, _L)])
            pltpu.async_copy(rcur, r_out.at[idxb.at[pl.ds(0, 1)]],
                             sem).wait()

        def fast_group(gbase):
            hs = [hcur[0, pl.ds(j * _L, _L)] for j in range(_D // _L)]
            accs = [jnp.zeros((_L,), _F32) for _ in range(_L)]
            for j in range(_D // _L):
                sl = pl.ds(j * _L, _L)
                for r in range(_L):
                    accs[r] = accs[r] + xbuf[gbase + r, sl] * hs[j]
            e_vec = _tree16(accs)
            m_grp = _lanemax(e_vec)
            p_vec = jnp.exp(e_vec - m_grp)
            s_grp = _lanesum(p_vec)
            m_v = mvec[...]
            m_new = jnp.maximum(m_v, m_grp)
            alpha = jnp.exp(m_v - m_new)
            beta = jnp.exp(m_grp - m_new)
            mvec[...] = m_new
            svec[...] = svec[...] * alpha + s_grp * beta
            ps = [_splat(p_vec, r) for r in range(_L)]
            ts = [jnp.zeros((_L,), _F32) for _ in range(_D // _L)]
            for r in range(_L):
                for j in range(_D // _L):
                    ts[j] = ts[j] + ps[r] * xbuf[gbase + r, pl.ds(j * _L, _L)]
            for j in range(_D // _L):
                sl = pl.ds(j * _L, _L)
                rcur[0, sl] = rcur[0, sl] * alpha + beta * ts[j]

        def slow_group(gbase, bv, b_cur):
            for r in range(_L):
                bi = bv[r]
                ch = bi != b_cur

                def on_new(b):
                    flush(b)
                    load_h(bi)
                    return bi

                b_cur = lax.cond(ch, on_new, lambda b: b, b_cur)
                chf = jnp.where(ch, 1.0, 0.0)
                m_old = mvec[...]
                m_v = m_old + chf * (_NEG - m_old)
                acc = jnp.zeros((_L,), _F32)
                for j in range(_D // _L):
                    sl = pl.ds(j * _L, _L)
                    acc = acc + xbuf[gbase + r, sl] * hcur[0, sl]
                e_v = _lanesum(acc)
                m_new = jnp.maximum(m_v, e_v)
                alpha = jnp.exp(m_v - m_new)
                p = jnp.exp(e_v - m_new)
                mvec[...] = m_new
                svec[...] = svec[...] * alpha + p
                for j in range(_D // _L):
                    sl = pl.ds(j * _L, _L)
                    rcur[0, sl] = rcur[0, sl] * alpha + p * xbuf[gbase + r, sl]
            return b_cur

        def group_body(g, b_cur):
            gbase = g * _L
            bv = bbuf[pl.ds(gbase, _L)]
            b_first = bv[0]
            b_last = bv[_L - 1]
            fast = jnp.logical_and(b_first == b_cur, b_last == b_cur)

            def do_fast(b):
                fast_group(gbase)
                return b

            def do_slow(b):
                return slow_group(gbase, bv, b)

            return lax.cond(fast, do_fast, do_slow, b_cur)

        def chunk_body(kk, b_cur):
            base = pl.multiple_of(start + kk * _CH, _CH)
            pltpu.sync_copy(bat_hbm.at[pl.ds(base, _CH)],
                            bbuf.at[pl.ds(0, _CH)])
            pltpu.sync_copy(x_hbm.at[pl.ds(base, _CH)], xbuf)
            return lax.fori_loop(0, _NG, group_body, b_cur)

        pltpu.sync_copy(bat_hbm.at[pl.ds(start, _CH)], bbuf.at[pl.ds(0, _CH)])
        b0 = bbuf[pl.ds(0, _L)][0]
        load_h(b0)
        b_cur = lax.fori_loop(0, _NCH, chunk_body, b0)

        flush(b_cur)
        pltpu.sync_copy(mloc, m_out.at[wid])
        pltpu.sync_copy(sloc, s_out.at[wid])

    return k(x, bat, h)


def kernel(x, batch, W_ih, W_hh, b_ih, b_hh):
    xsc = x[_NTC:]
    batsc = batch[_NTC:]
    xtc = jnp.pad(x[:_NTC], ((0, _NPTC - _NTC), (0, 0)))
    bat3 = jnp.pad(batch[:_NTC], (0, _NPTC - _NTC),
                   constant_values=_B).reshape(_NBLK, _R, 1)
    bias = (b_ih + b_hh).reshape(1, 4 * _D)
    h = jnp.zeros((_B, _D), _F32)
    c = jnp.zeros((_B, _D), _F32)
    mp = jnp.full((_NW, _B), _NEG, _F32)
    sp = jnp.zeros((_NW, _B), _F32)
    rp = jnp.zeros((_NW * _B + 8, _D), _F32)
    mt = jnp.full((1, _B), _NEG, _F32)
    st = jnp.zeros((1, _B), _F32)
    rt = jnp.zeros((_B, _D), _F32)
    q_star = None
    for _ in range(3):
        h, c, q_star = _merge_lstm(mp, sp, rp[:_NW * _B], mt, st, rt,
                                   h, c, W_ih, W_hh, bias)
        mp, sp, rp = _sc_attn(xsc, batsc, h)
        mt, st, rt = _tc_attn(bat3, xtc, h)
    _, _, q_star = _merge_lstm(mp, sp, rp[:_NW * _B], mt, st, rt,
                               h, c, W_ih, W_hh, bias)
    return q_star


# speedup vs baseline: 3.3063x; 1.0074x over previous
import functools

import jax
import jax.numpy as jnp
from jax import lax
from jax.experimental import pallas as pl
from jax.experimental.pallas import tpu as pltpu
from jax.experimental.pallas import tpu_sc as plsc

_N = 50000
_D = 256
_B = 256

_NW = 32
_C = 352
_CH = 352
_NCH = _C // _CH
_L = 16
_NG = _CH // _L
_NSC = _NW * _C
_NTC = _N - _NSC
_R = 2048
_NBLK = (_NTC + _R - 1) // _R
_NPTC = _NBLK * _R

_F32 = jnp.float32
_NEG = -1e30


def _col(v):
    ib = (lax.broadcasted_iota(jnp.int32, (_B, _B), 0) ==
          lax.broadcasted_iota(jnp.int32, (_B, _B), 1))
    return jnp.sum(jnp.where(ib, v, 0.0), axis=1, keepdims=True)


def _tca_body(bat_ref, x_ref, h_ref, m_out, s_out, r_out, m_s, s_s, r_s):
    blk = pl.program_id(0)

    @pl.when(blk == 0)
    def _init():
        m_s[...] = jnp.full_like(m_s, _NEG)
        s_s[...] = jnp.zeros_like(s_s)
        r_s[...] = jnp.zeros_like(r_s)

    bat = bat_ref[0]
    iota_b = lax.broadcasted_iota(jnp.int32, (_R, _B), 1)
    pmask = bat == iota_b
    pf = pmask.astype(_F32)
    h = h_ref[...]
    h_hi = h.astype(jnp.bfloat16).astype(_F32)
    h_lo = h - h_hi
    qg = (lax.dot_general(pf, h_hi, (((1,), (0,)), ((), ())),
                          preferred_element_type=_F32)
          + lax.dot_general(pf, h_lo, (((1,), (0,)), ((), ())),
                            preferred_element_type=_F32))
    xb = x_ref[...]
    e = jnp.sum(xb * qg, axis=1, keepdims=True)
    em = jnp.where(pmask, e, _NEG)
    mblk = jnp.max(em, axis=0, keepdims=True)
    m_old = m_s[...]
    m_new = jnp.maximum(m_old, mblk)
    scale = jnp.exp(m_old - m_new)
    gm = jnp.sum(jnp.where(pmask, m_new, 0.0), axis=1, keepdims=True)
    ex = jnp.exp(e - gm)
    pw = pf * ex
    sblk = jnp.sum(pw, axis=0, keepdims=True)
    m_s[...] = m_new
    s_s[...] = s_s[...] * scale + sblk
    scale_col = _col(scale)
    racc = lax.dot_general(pw, xb, (((0,), (0,)), ((), ())),
                           preferred_element_type=_F32)
    r_s[...] = r_s[...] * scale_col + racc

    @pl.when(blk == _NBLK - 1)
    def _fin():
        m_out[...] = m_s[...]
        s_out[...] = s_s[...]
        r_out[...] = r_s[...]


def _tc_attn(bat3, xtc, h):
    return pl.pallas_call(
        _tca_body,
        grid=(_NBLK,),
        in_specs=[
            pl.BlockSpec((1, _R, 1), lambda b: (b, 0, 0)),
            pl.BlockSpec((_R, _D), lambda b: (b, 0)),
            pl.BlockSpec((_B, _D), lambda b: (0, 0)),
        ],
        out_specs=[
            pl.BlockSpec((1, _B), lambda b: (0, 0)),
            pl.BlockSpec((1, _B), lambda b: (0, 0)),
            pl.BlockSpec((_B, _D), lambda b: (0, 0)),
        ],
        out_shape=[
            jax.ShapeDtypeStruct((1, _B), _F32),
            jax.ShapeDtypeStruct((1, _B), _F32),
            jax.ShapeDtypeStruct((_B, _D), _F32),
        ],
        scratch_shapes=[
            pltpu.VMEM((1, _B), _F32),
            pltpu.VMEM((1, _B), _F32),
            pltpu.VMEM((_B, _D), _F32),
        ],
    )(bat3, xtc, h)


def _ml_body(m_ref, s_ref, r_ref, mt_ref, st_ref, rt_ref, h_ref, c_ref,
             wih_ref, whh_ref, bias_ref, hn_ref, cn_ref, q_ref):
    mp = m_ref[...]
    sp = s_ref[...]
    mt = mt_ref[...]
    mstar = jnp.maximum(jnp.max(mp, axis=0, keepdims=True), mt)
    valid = mp > -1e29
    w = jnp.where(valid, jnp.exp(mp - mstar), 0.0)
    wt = jnp.where(mt > -1e29, jnp.exp(mt - mstar), 0.0)
    sstar = jnp.sum(w * sp, axis=0, keepdims=True) + wt * st_ref[...]
    rstar = _col(wt) * rt_ref[...]
    for i in range(_NW):
        wcol = _col(w[i:i + 1])
        ri = r_ref[pl.ds(i * _B, _B), :]
        ri = jnp.where(wcol > 0.0, ri, 0.0)
        rstar = rstar + wcol * ri
    r_fin = rstar / (_col(sstar) + 1e-16)
    h = h_ref[...]
    q_star = jnp.concatenate([h, r_fin], axis=1)
    g = (lax.dot_general(q_star, wih_ref[...], (((1,), (1,)), ((), ())),
                         preferred_element_type=_F32)
         + lax.dot_general(h, whh_ref[...], (((1,), (1,)), ((), ())),
                           preferred_element_type=_F32)
         + bias_ref[...])
    gi = jax.nn.sigmoid(g[:, :_D])
    gf = jax.nn.sigmoid(g[:, _D:2 * _D])
    gg = jnp.tanh(g[:, 2 * _D:3 * _D])
    go = jax.nn.sigmoid(g[:, 3 * _D:])
    c_new = gf * c_ref[...] + gi * gg
    h_new = go * jnp.tanh(c_new)
    hn_ref[...] = h_new
    cn_ref[...] = c_new
    q_ref[...] = q_star


def _merge_lstm(mp, sp, rp, mt, st, rt, h, c, W_ih, W_hh, bias):
    return pl.pallas_call(
        _ml_body,
        out_shape=[
            jax.ShapeDtypeStruct((_B, _D), _F32),
            jax.ShapeDtypeStruct((_B, _D), _F32),
            jax.ShapeDtypeStruct((_B, 2 * _D), _F32),
        ],
    )(mp, sp, rp, mt, st, rt, h, c, W_ih, W_hh, bias)


def _swap(v, k):
    iota = lax.broadcasted_iota(jnp.int32, (_L,), 0)
    return v.at[jnp.bitwise_xor(iota, k)].get(mode="promise_in_bounds")


def _lanesum(v):
    for k in (8, 4, 2, 1):
        v = v + _swap(v, k)
    return v


def _lanemax(v):
    for k in (8, 4, 2, 1):
        v = jnp.maximum(v, _swap(v, k))
    return v


def _tree16(vs):
    cur = list(vs)
    for k in (8, 4, 2, 1):
        n = len(cur) // 2
        iota = lax.broadcasted_iota(jnp.int32, (_L,), 0)
        sel = (iota & k) == 0
        cur = [jnp.where(sel, cur[i] + _swap(cur[i], k),
                         cur[i + n] + _swap(cur[i + n], k))
               for i in range(n)]
    return cur[0]


def _splat(v, lane):
    return v.at[jnp.full((_L,), lane, jnp.int32)].get(
        mode="promise_in_bounds")


def _sc_attn(x, bat, h):
    mesh = plsc.VectorSubcoreMesh(core_axis_name="c", subcore_axis_name="s")

    @functools.partial(
        pl.kernel,
        mesh=mesh,
        out_type=[
            jax.ShapeDtypeStruct((_NW, _B), _F32),
            jax.ShapeDtypeStruct((_NW, _B), _F32),
            jax.ShapeDtypeStruct((_NW * _B + 8, _D), _F32),
        ],
        scratch_types=[
            pltpu.VMEM((_CH, _D), _F32),
            pltpu.VMEM((_CH + _L,), jnp.int32),
            pltpu.VMEM((_L, _D), _F32),
            pltpu.VMEM((1, _D), _F32),
            pltpu.VMEM((_L,), _F32),
            pltpu.VMEM((_L,), _F32),
            pltpu.VMEM((_B,), _F32),
            pltpu.VMEM((_B,), _F32),
            pltpu.VMEM((_L,), jnp.int32),
            pltpu.SemaphoreType.DMA,
        ],
    )
    def k(x_hbm, bat_hbm, h_hbm, m_out, s_out, r_out,
          xbuf, bbuf, hcur, rcur, mvec, svec, mloc, sloc, idxb, sem):
        cid = lax.axis_index("c")
        sid = lax.axis_index("s")
        wid = sid * 2 + cid
        start = wid * _C
        iota = lax.broadcasted_iota(jnp.int32, (_L,), 0)

        for j in range(_B // _L):
            mloc[pl.ds(j * _L, _L)] = jnp.full((_L,), _NEG, _F32)
            sloc[pl.ds(j * _L, _L)] = jnp.zeros((_L,), _F32)
        for j in range(_D // _L):
            rcur[0, pl.ds(j * _L, _L)] = jnp.zeros((_L,), _F32)
        mvec[...] = jnp.full((_L,), _NEG, _F32)
        svec[...] = jnp.zeros((_L,), _F32)

        def load_h(b):
            bsafe = jnp.minimum(b, _B - 1)
            pltpu.async_copy(h_hbm.at[jnp.full((_L,), bsafe, jnp.int32)],
                             hcur, sem).wait()

        def flush(b_old):
            m_old = mvec[...]
            s_old = svec[...]
            for j in range(_B // _L):
                sl = pl.ds(j * _L, _L)
                sel = (iota + j * _L) == b_old
                mloc[sl] = jnp.where(sel, m_old, mloc[sl])
                sloc[sl] = jnp.where(sel, s_old, sloc[sl])
            ridx = jnp.where(b_old < _B, wid * _B + b_old, _NW * _B)
            idxb[pl.ds(0, _L)] = jnp.where(iota == 0, ridx,
                                           idxb[pl.ds(0, _L)])
            pltpu.async_copy(rcur, r_out.at[idxb.at[pl.ds(0, 1)]],
                             sem).wait()

        def fast_group(gbase):
            hs = [hcur[0, pl.ds(j * _L, _L)] for j in range(_D // _L)]
            accs = [jnp.zeros((_L,), _F32) for _ in range(_L)]
            for j in range(_D // _L):
                sl = pl.ds(j * _L, _L)
                for r in range(_L):
                    accs[r] = accs[r] + xbuf[gbase + r, sl] * hs[j]
            e_vec = _tree16(accs)
            m_grp = _lanemax(e_vec)
            p_vec = jnp.exp(e_vec - m_grp)
            s_grp = _lanesum(p_vec)
            m_v = mvec[...]
            m_new = jnp.maximum(m_v, m_grp)
            alpha = jnp.exp(m_v - m_new)
            beta = jnp.exp(m_grp - m_new)
            mvec[...] = m_new
            svec[...] = svec[...] * alpha + s_grp * beta
            ps = [_splat(p_vec, r) for r in range(_L)]
            ts = [jnp.zeros((_L,), _F32) for _ in range(_D // _L)]
            for r in range(_L):
                for j in range(_D // _L):
                    ts[j] = ts[j] + ps[r] * xbuf[gbase + r, pl.ds(j * _L, _L)]
            for j in range(_D // _L):
                sl = pl.ds(j * _L, _L)
                rcur[0, sl] = rcur[0, sl] * alpha + beta * ts[j]

        def slow_group(gbase, bv, b_cur):
            for r in range(_L):
                bi = bv[r]
                ch = bi != b_cur

                def on_new(b):
                    flush(b)
                    load_h(bi)
                    return bi

                b_cur = lax.cond(ch, on_new, lambda b: b, b_cur)
                chf = jnp.where(ch, 1.0, 0.0)
                m_old = mvec[...]
                m_v = m_old + chf * (_NEG - m_old)
                acc = jnp.zeros((_L,), _F32)
                for j in range(_D // _L):
                    sl = pl.ds(j * _L, _L)
                    acc = acc + xbuf[gbase + r, sl] * hcur[0, sl]
                e_v = _lanesum(acc)
                m_new = jnp.maximum(m_v, e_v)
                alpha = jnp.exp(m_v - m_new)
                p = jnp.exp(e_v - m_new)
                mvec[...] = m_new
                svec[...] = svec[...] * alpha + p
                for j in range(_D // _L):
                    sl = pl.ds(j * _L, _L)
                    rcur[0, sl] = rcur[0, sl] * alpha + p * xbuf[gbase + r, sl]
            return b_cur

        def group_body(g, b_cur):
            gbase = g * _L
            bv = bbuf[pl.ds(gbase, _L)]
            b_first = bv[0]
            b_last = bv[_L - 1]
            fast = jnp.logical_and(b_first == b_cur, b_last == b_cur)

            def do_fast(b):
                fast_group(gbase)
                return b

            def do_slow(b):
                return slow_group(gbase, bv, b)

            return lax.cond(fast, do_fast, do_slow, b_cur)

        def chunk_body(kk, b_cur):
            base = pl.multiple_of(start + kk * _CH, _CH)
            pltpu.sync_copy(bat_hbm.at[pl.ds(base, _CH)],
                            bbuf.at[pl.ds(0, _CH)])
            pltpu.sync_copy(x_hbm.at[pl.ds(base, _CH)], xbuf)
            return lax.fori_loop(0, _NG, group_body, b_cur)

        pltpu.sync_copy(bat_hbm.at[pl.ds(start, _CH)], bbuf.at[pl.ds(0, _CH)])
        b0 = bbuf[pl.ds(0, _L)][0]
        load_h(b0)
        b_cur = lax.fori_loop(0, _NCH, chunk_body, b0)

        flush(b_cur)
        pltpu.sync_copy(mloc, m_out.at[wid])
        pltpu.sync_copy(sloc, s_out.at[wid])

    return k(x, bat, h)


def kernel(x, batch, W_ih, W_hh, b_ih, b_hh):
    xsc = x[_NTC:]
    batsc = batch[_NTC:]
    xtc = jnp.pad(x[:_NTC], ((0, _NPTC - _NTC), (0, 0)))
    bat3 = jnp.pad(batch[:_NTC], (0, _NPTC - _NTC),
                   constant_values=_B).reshape(_NBLK, _R, 1)
    bias = (b_ih + b_hh).reshape(1, 4 * _D)
    h = jnp.zeros((_B, _D), _F32)
    c = jnp.zeros((_B, _D), _F32)
    mp = jnp.full((_NW, _B), _NEG, _F32)
    sp = jnp.zeros((_NW, _B), _F32)
    rp = jnp.zeros((_NW * _B + 8, _D), _F32)
    mt = jnp.full((1, _B), _NEG, _F32)
    st = jnp.zeros((1, _B), _F32)
    rt = jnp.zeros((_B, _D), _F32)
    q_star = None
    for _ in range(3):
        h, c, q_star = _merge_lstm(mp, sp, rp[:_NW * _B], mt, st, rt,
                                   h, c, W_ih, W_hh, bias)
        mp, sp, rp = _sc_attn(xsc, batsc, h)
        mt, st, rt = _tc_attn(bat3, xtc, h)
    _, _, q_star = _merge_lstm(mp, sp, rp[:_NW * _B], mt, st, rt,
                               h, c, W_ih, W_hh, bias)
    return q_star
